# baseline (speedup 1.0000x reference)
import functools

import jax
import jax.numpy as jnp
from jax import lax
from jax.experimental import pallas as pl
from jax.experimental.pallas import tpu as pltpu
from jax.experimental.pallas import tpu_sc as plsc

_AUDIO_VOCAB = 2051
_NUM_CB = 32
_LANES = 16

_NUM_CORES = 2
_NUM_SUBCORES = 16
_NW = _NUM_CORES * _NUM_SUBCORES


def _make_gather(S, D):
    assert S % _NW == 0
    n_pos = S // _NW
    CH = 32
    assert n_pos % CH == 0
    n_h = n_pos // CH
    assert n_h == 2
    C1 = _NUM_CB + 1

    mesh = plsc.VectorSubcoreMesh(core_axis_name="c", subcore_axis_name="s")

    @functools.partial(
        pl.kernel,
        mesh=mesh,
        out_type=jax.ShapeDtypeStruct((C1, S, D), jnp.float32),
        scratch_types=[
            pltpu.VMEM((C1, 2 * n_pos), jnp.int32),
            pltpu.VMEM((2, CH), jnp.int32),
            pltpu.VMEM((2, CH, D), jnp.float32),
            pltpu.SemaphoreType.DMA,
            pltpu.SemaphoreType.DMA,
        ],
    )
    def gather_kernel(tok_hbm, text_hbm, audio_hbm, out_hbm,
                      tok_v, idx_v, buf_v, sem0, sem1):
        sems = (sem0, sem1)
        cid = lax.axis_index("c")
        sid = lax.axis_index("s")
        wid = sid * _NUM_CORES + cid
        base = wid * n_pos

        pair_base = (wid // 2) * (2 * n_pos)
        tok_off = (wid % 2) * n_pos
        pltpu.sync_copy(tok_hbm.at[:, pl.ds(pair_base, 2 * n_pos)], tok_v)

        def fill_and_start(j, h, slot):
            off = jnp.where(j < _NUM_CB, j * _AUDIO_VOCAB, 0).astype(jnp.int32)
            lo = tok_v[j, pl.ds(tok_off + h * CH, _LANES)] + off
            hi = tok_v[j, pl.ds(tok_off + h * CH + _LANES, _LANES)] + off
            idx_v[slot, pl.ds(0, _LANES)] = lo
            idx_v[slot, pl.ds(_LANES, _LANES)] = hi

            @pl.when(j < _NUM_CB)
            def _():
                pltpu.async_copy(
                    audio_hbm.at[idx_v.at[slot]], buf_v.at[slot], sems[slot])

            @pl.when(j >= _NUM_CB)
            def _():
                pltpu.async_copy(
                    text_hbm.at[idx_v.at[slot]], buf_v.at[slot], sems[slot])

        def drain(slot):
            pltpu.make_async_copy(
                audio_hbm.at[idx_v.at[slot]], buf_v.at[slot],
                sems[slot]).wait()

        def write(j, h, slot):
            pltpu.sync_copy(
                buf_v.at[slot], out_hbm.at[j, pl.ds(base + h * CH, CH)])

        fill_and_start(0, 0, 0)
        fill_and_start(0, 1, 1)

        def step(j, _):
            drain(0)
            write(j, 0, 0)

            @pl.when(j + 1 < C1)
            def _():
                fill_and_start(j + 1, 0, 0)

            drain(1)
            write(j, 1, 1)

            @pl.when(j + 1 < C1)
            def _():
                fill_and_start(j + 1, 1, 1)

            return _

        lax.fori_loop(0, C1, step, None)

    return gather_kernel


def kernel(tokens, text_table, audio_table):
    B, S, C1 = tokens.shape
    D = text_table.shape[1]
    tok_t = tokens.reshape(S, C1).astype(jnp.int32).T
    out = _make_gather(S, D)(tok_t, text_table, audio_table)
    return out.transpose(1, 0, 2).reshape(B, S, C1, D)
